# baseline (speedup 1.0000x reference)
import jax
import jax.numpy as jnp
from jax.experimental import pallas as pl
from jax.experimental.pallas import tpu as pltpu

_N = 10000
_D = 128
_K = 16
_NP = 10240
_RQ = 256
_W = 128
_G = _NP // _W
_J = 5
_S = 1
_Q = 4
_SW = _NP // _S


def _dist_slab(xq, xall, sqq, i, s):
    xs = xall[s * _SW:(s + 1) * _SW, :]
    prod = jax.lax.dot_general(
        xs, xq, (((1,), (1,)), ((), ())),
        preferred_element_type=jnp.float32)
    sqc = jnp.sum(xs * xs, axis=1, keepdims=True)
    sqc = jnp.where(
        jax.lax.broadcasted_iota(jnp.int32, (_SW, 1), 0) + s * _SW >= _N,
        1e30, sqc)
    dist = sqc + sqq - 2.0 * prod
    srow = jax.lax.broadcasted_iota(jnp.int32, (_SW, _RQ), 0) + s * _SW
    qcol = jax.lax.broadcasted_iota(jnp.int32, (_SW, _RQ), 1) + i * _RQ
    return jnp.where(srow == qcol, dist + 1e9, dist)


def _knn_body(xq_ref, x_ref, sqq_ref, idx_ref, dist_ref):
    i = pl.program_id(0)
    xq = xq_ref[...]
    xall = x_ref[...]
    sqq = sqq_ref[...]

    gs = _SW // _W
    si3 = jax.lax.broadcasted_iota(jnp.int32, (gs, _W, _RQ), 1)
    cvals = [[] for _ in range(_J)]
    cidxs = [[] for _ in range(_J)]
    for s in range(_S):
        d3 = _dist_slab(xq, xall, sqq, i, s).reshape(gs, _W, _RQ)
        gbase = (jax.lax.broadcasted_iota(jnp.int32, (gs, _RQ), 0)
                 + s * gs) * _W
        for j in range(_J):
            m = jnp.min(d3, axis=1)
            a = jnp.argmin(d3, axis=1).astype(jnp.int32)
            cvals[j].append(m)
            cidxs[j].append(a + gbase)
            if j < _J - 1:
                d3 = jnp.where(si3 == a[:, None, :], jnp.inf, d3)
    cvals = [jnp.concatenate(v, axis=0) for v in cvals]
    cidxs = [jnp.concatenate(v, axis=0) for v in cidxs]

    cv = jnp.stack(cvals, axis=1).reshape(_G * _J, _RQ)
    civ = jnp.stack(cidxs, axis=1).reshape(_G * _J, _RQ)
    qrows = _G * _J // _Q
    bad = jnp.zeros((), jnp.bool_)
    qvs, qis = [], []
    for q in range(_Q):
        qv = cv[q * qrows:(q + 1) * qrows, :]
        qi = civ[q * qrows:(q + 1) * qrows, :]
        qrow = jax.lax.broadcasted_iota(jnp.int32, (qrows, _RQ), 0)
        deepest = (qrow % _J) == (_J - 1)
        drained = jnp.zeros((qrows, _RQ), jnp.bool_)
        for k in range(_K):
            p = jnp.argmin(qv.reshape(1, qrows, _RQ), axis=1).astype(jnp.int32)
            sel = qrow == p
            qvs.append(jnp.min(qv.reshape(1, qrows, _RQ), axis=1))
            qis.append(jnp.sum(
                jnp.where(sel, qi, 0).reshape(1, qrows, _RQ), axis=1))
            drained = drained | (sel & deepest)
            qv = jnp.where(sel, jnp.inf, qv)
        bad = bad | jnp.any(drained)
    fv = jnp.concatenate(qvs, axis=0)
    fi = jnp.concatenate(qis, axis=0)
    frow = jax.lax.broadcasted_iota(jnp.int32, (_Q * _K, _RQ), 0)
    for k in range(_K):
        p = jnp.argmin(fv.reshape(1, _Q * _K, _RQ), axis=1).astype(jnp.int32)
        sel = frow == p
        idx_ref[k:k + 1, :] = jnp.sum(
            jnp.where(sel, fi, 0).reshape(1, _Q * _K, _RQ), axis=1)
        fv = jnp.where(sel, jnp.inf, fv)

    @pl.when(bad)
    def _repair():
        dist_ref[...] = jnp.concatenate(
            [_dist_slab(xq, xall, sqq, i, s) for s in range(_S)], axis=0)
        srow = jax.lax.broadcasted_iota(jnp.int32, (_NP, _RQ), 0)
        for k in range(_K):
            cur = dist_ref[...]
            a = jnp.argmin(cur.reshape(1, _NP, _RQ), axis=1).astype(jnp.int32)
            idx_ref[k:k + 1, :] = a
            dist_ref[...] = jnp.where(srow == a, jnp.inf, cur)


def kernel(node_feat, dynamicVariable, edge_index):
    x = dynamicVariable
    x_pad = jnp.concatenate(
        [x, jnp.zeros((_NP - _N, _D), jnp.float32)], axis=0)
    sq = jnp.sum(x * x, axis=1)
    sqq_pad = jnp.concatenate(
        [sq, jnp.zeros((_NP - _N,), jnp.float32)])[None, :]
    idx_t = pl.pallas_call(
        _knn_body,
        grid=(_NP // _RQ,),
        in_specs=[
            pl.BlockSpec((_RQ, _D), lambda i: (i, 0)),
            pl.BlockSpec((_NP, _D), lambda i: (0, 0)),
            pl.BlockSpec((1, _RQ), lambda i: (0, i)),
        ],
        out_specs=pl.BlockSpec((_K, _RQ), lambda i: (0, i)),
        out_shape=jax.ShapeDtypeStruct((_K, _NP), jnp.int32),
        scratch_shapes=[pltpu.VMEM((_NP, _RQ), jnp.float32)],
    )(x_pad, x_pad, sqq_pad)

    idx = idx_t[:, :_N].T
    src = idx.reshape(-1)
    dst = jnp.repeat(jnp.arange(_N, dtype=src.dtype), _K)
    new_edges = jnp.stack([src, dst]).astype(jnp.int64)
    skip = jnp.allclose(node_feat, dynamicVariable)
    out_feat = jnp.where(skip, node_feat, dynamicVariable)
    out_edges = jnp.where(skip, edge_index, new_edges)
    return out_feat, out_edges


# speedup vs baseline: 3.0542x; 3.0542x over previous
import jax
import jax.numpy as jnp
from jax.experimental import pallas as pl
from jax.experimental.pallas import tpu as pltpu

_N = 10000
_D = 128
_K = 16
_NP = 10240
_RQ = 256
_W = 128
_G = _NP // _W
_J = 5


def _knn_body(xq_ref, x_ref, sqq_ref, idx_ref, dist_ref):
    i = pl.program_id(0)
    xq = xq_ref[...]
    xall = x_ref[...]
    prod = jax.lax.dot_general(
        xall, xq, (((1,), (1,)), ((), ())),
        preferred_element_type=jnp.float32)
    sqc = jnp.sum(xall * xall, axis=1, keepdims=True)
    sqc = jnp.where(
        jax.lax.broadcasted_iota(jnp.int32, (_NP, 1), 0) >= _N, 1e30, sqc)
    dist = sqc + sqq_ref[...] - 2.0 * prod
    srow = jax.lax.broadcasted_iota(jnp.int32, (_NP, _RQ), 0)
    qcol = jax.lax.broadcasted_iota(jnp.int32, (_NP, _RQ), 1) + i * _RQ
    dist = jnp.where(srow == qcol, dist + 1e9, dist)
    dist_ref[...] = dist

    d3 = dist.reshape(_G, _W, _RQ)
    gbase = jax.lax.broadcasted_iota(jnp.int32, (_G, _RQ), 0) * _W
    si3 = jax.lax.broadcasted_iota(jnp.int32, (_G, _W, _RQ), 1)
    cvals, cidxs = [], []
    for j in range(_J):
        m = jnp.min(d3, axis=1)
        a = jnp.argmin(d3, axis=1).astype(jnp.int32)
        cvals.append(m)
        cidxs.append(a + gbase)
        if j < _J - 1:
            d3 = jnp.where(si3 == a[:, None, :], jnp.inf, d3)

    cv = jnp.stack(cvals, axis=1).reshape(_G * _J, _RQ)
    civ = jnp.stack(cidxs, axis=1).reshape(_G * _J, _RQ)
    crow = jax.lax.broadcasted_iota(jnp.int32, (_G * _J, _RQ), 0)
    deepest = (crow % _J) == (_J - 1)
    drained = jnp.zeros((_G * _J, _RQ), jnp.bool_)
    for k in range(_K):
        p = jnp.argmin(cv, axis=0).astype(jnp.int32)
        sel = crow == p[None, :]
        idx_ref[k:k + 1, :] = jnp.sum(jnp.where(sel, civ, 0), axis=0,
                                      keepdims=True)
        drained = drained | (sel & deepest)
        cv = jnp.where(sel, jnp.inf, cv)
    bad = jnp.any(drained)

    @pl.when(bad)
    def _repair():
        for k in range(_K):
            cur = dist_ref[...]
            a = jnp.argmin(cur, axis=0).astype(jnp.int32)
            idx_ref[k:k + 1, :] = a[None, :]
            dist_ref[...] = jnp.where(srow == a[None, :], jnp.inf, cur)


def kernel(node_feat, dynamicVariable, edge_index):
    x = dynamicVariable
    x_pad = jnp.concatenate(
        [x, jnp.zeros((_NP - _N, _D), jnp.float32)], axis=0)
    sq = jnp.sum(x * x, axis=1)
    sqq_pad = jnp.concatenate(
        [sq, jnp.zeros((_NP - _N,), jnp.float32)])[None, :]
    idx_t = pl.pallas_call(
        _knn_body,
        grid=(_NP // _RQ,),
        in_specs=[
            pl.BlockSpec((_RQ, _D), lambda i: (i, 0)),
            pl.BlockSpec((_NP, _D), lambda i: (0, 0)),
            pl.BlockSpec((1, _RQ), lambda i: (0, i)),
        ],
        out_specs=pl.BlockSpec((_K, _RQ), lambda i: (0, i)),
        out_shape=jax.ShapeDtypeStruct((_K, _NP), jnp.int32),
        scratch_shapes=[pltpu.VMEM((_NP, _RQ), jnp.float32)],
    )(x_pad, x_pad, sqq_pad)

    idx = idx_t[:, :_N].T
    src = idx.reshape(-1)
    dst = jnp.repeat(jnp.arange(_N, dtype=src.dtype), _K)
    new_edges = jnp.stack([src, dst]).astype(jnp.int64)
    skip = jnp.allclose(node_feat, dynamicVariable)
    out_feat = jnp.where(skip, node_feat, dynamicVariable)
    out_edges = jnp.where(skip, edge_index, new_edges)
    return out_feat, out_edges


# speedup vs baseline: 3.0902x; 1.0118x over previous
import jax
import jax.numpy as jnp
from jax.experimental import pallas as pl
from jax.experimental.pallas import tpu as pltpu

_N = 10000
_D = 128
_K = 16
_NP = 10240
_RQ = 256
_W = 128
_G = _NP // _W
_J = 5


def _knn_body(xq_ref, x_ref, sqq_ref, sqc_ref, idx_ref, dist_ref):
    i = pl.program_id(0)
    xq = xq_ref[...]
    xall = x_ref[...]
    prod = jax.lax.dot_general(
        xall, xq, (((1,), (1,)), ((), ())),
        preferred_element_type=jnp.float32)
    dist = sqc_ref[...] + sqq_ref[...] - 2.0 * prod
    srow = jax.lax.broadcasted_iota(jnp.int32, (_NP, _RQ), 0)
    qcol = jax.lax.broadcasted_iota(jnp.int32, (_NP, _RQ), 1) + i * _RQ
    dist = jnp.where(srow == qcol, dist + 1e9, dist)
    dist_ref[...] = dist

    d3 = dist.reshape(_G, _W, _RQ)
    gbase = jax.lax.broadcasted_iota(jnp.int32, (_G, _RQ), 0) * _W
    si3 = jax.lax.broadcasted_iota(jnp.int32, (_G, _W, _RQ), 1)
    cvals, cidxs = [], []
    for j in range(_J):
        m = jnp.min(d3, axis=1)
        a = jnp.argmin(d3, axis=1).astype(jnp.int32)
        cvals.append(m)
        cidxs.append(a + gbase)
        if j < _J - 1:
            d3 = jnp.where(si3 == a[:, None, :], jnp.inf, d3)

    cv = jnp.stack(cvals, axis=1).reshape(_G * _J, _RQ)
    civ = jnp.stack(cidxs, axis=1).reshape(_G * _J, _RQ)
    crow = jax.lax.broadcasted_iota(jnp.int32, (_G * _J, _RQ), 0)
    deepest = (crow % _J) == (_J - 1)
    drained = jnp.zeros((_G * _J, _RQ), jnp.bool_)
    for k in range(_K):
        p = jnp.argmin(cv, axis=0).astype(jnp.int32)
        sel = crow == p[None, :]
        idx_ref[k:k + 1, :] = jnp.sum(jnp.where(sel, civ, 0), axis=0,
                                      keepdims=True)
        drained = drained | (sel & deepest)
        cv = jnp.where(sel, jnp.inf, cv)
    bad = jnp.any(drained)

    @pl.when(bad)
    def _repair():
        for k in range(_K):
            cur = dist_ref[...]
            a = jnp.argmin(cur, axis=0).astype(jnp.int32)
            idx_ref[k:k + 1, :] = a[None, :]
            dist_ref[...] = jnp.where(srow == a[None, :], jnp.inf, cur)


def kernel(node_feat, dynamicVariable, edge_index):
    x = dynamicVariable
    x_pad = jnp.concatenate(
        [x, jnp.zeros((_NP - _N, _D), jnp.float32)], axis=0)
    sq = jnp.sum(x * x, axis=1)
    sqq_pad = jnp.concatenate(
        [sq, jnp.zeros((_NP - _N,), jnp.float32)])[None, :]
    sqc_pad = jnp.concatenate(
        [sq, jnp.full((_NP - _N,), 1e30, jnp.float32)])[:, None]
    idx_t = pl.pallas_call(
        _knn_body,
        grid=(_NP // _RQ,),
        in_specs=[
            pl.BlockSpec((_RQ, _D), lambda i: (i, 0)),
            pl.BlockSpec((_NP, _D), lambda i: (0, 0)),
            pl.BlockSpec((1, _RQ), lambda i: (0, i)),
            pl.BlockSpec((_NP, 1), lambda i: (0, 0)),
        ],
        out_specs=pl.BlockSpec((_K, _RQ), lambda i: (0, i)),
        out_shape=jax.ShapeDtypeStruct((_K, _NP), jnp.int32),
        scratch_shapes=[pltpu.VMEM((_NP, _RQ), jnp.float32)],
    )(x_pad, x_pad, sqq_pad, sqc_pad)

    idx = idx_t[:, :_N].T
    src = idx.reshape(-1)
    dst = jnp.repeat(jnp.arange(_N, dtype=src.dtype), _K)
    new_edges = jnp.stack([src, dst]).astype(jnp.int64)
    skip = jnp.allclose(node_feat, dynamicVariable)
    out_feat = jnp.where(skip, node_feat, dynamicVariable)
    out_edges = jnp.where(skip, edge_index, new_edges)
    return out_feat, out_edges
